# Initial kernel scaffold; baseline (speedup 1.0000x reference)
import jax
import jax.numpy as jnp
from jax.experimental import pallas as pl


def kernel(x, edge_index, edge_attr, a2b, b2a, b2revb, W, b):
    raise NotImplementedError("write your pallas kernel here")


# speedup vs baseline: 1.6645x; 1.6645x over previous
import functools

import jax
import jax.numpy as jnp
from jax import lax
from jax.experimental import pallas as pl
from jax.experimental.pallas import tpu as pltpu
from jax.experimental.pallas import tpu_sc as plsc

N = 10000
E = 160000
D = 256
MAX_NB = 16

NC = 2
NS = 16
NW = NC * NS

N_PAD = 10240
APW = N_PAD // NW
CA = 8
EPW = E // NW
CE = 40

_MESH = plsc.VectorSubcoreMesh(core_axis_name="c", subcore_axis_name="s",
                               num_cores=NC, num_subcores=NS)


def _wid():
    return lax.axis_index("s") * NC + lax.axis_index("c")


def _pipelined(nchunks, issue, process):
    issue(0, 0)
    if nchunks > 1:
        issue(1, 1)
    npairs = (nchunks + 1) // 2

    def body(j, carry):
        i0 = 2 * j
        process(i0, 0)

        @pl.when(i0 + 2 < nchunks)
        def _():
            issue(i0 + 2, 0)

        @pl.when(i0 + 1 < nchunks)
        def _():
            process(i0 + 1, 1)

        @pl.when(i0 + 3 < nchunks)
        def _():
            issue(i0 + 3, 1)

        return carry

    lax.fori_loop(0, npairs, body, 0, unroll=False)


@functools.partial(
    pl.kernel,
    out_type=jax.ShapeDtypeStruct((N_PAD, D), jnp.float32),
    mesh=_MESH,
    scratch_types=[
        pltpu.VMEM((APW * MAX_NB,), jnp.int32),
        pltpu.VMEM((2, CA * MAX_NB, D), jnp.float32),
        pltpu.VMEM((2, CA, D), jnp.float32),
        pltpu.SemaphoreType.DMA,
        pltpu.SemaphoreType.DMA,
        pltpu.SemaphoreType.DMA,
        pltpu.SemaphoreType.DMA,
    ],
)
def _aggregate(a2b_hbm, edge_attr_hbm, amsg_hbm, idx_v, rows_v, acc_v,
               sem0, sem1, osem0, osem1):
    wid = _wid()
    abase = wid * APW
    pltpu.sync_copy(a2b_hbm.at[pl.ds(abase * MAX_NB, APW * MAX_NB)], idx_v)
    sems = (sem0, sem1)
    osems = (osem0, osem1)
    nchunks = APW // CA

    def issue(i, slot):
        pltpu.async_copy(
            edge_attr_hbm.at[idx_v.at[pl.ds(i * CA * MAX_NB, CA * MAX_NB)]],
            rows_v.at[slot], sems[slot])

    def process(i, slot):
        pltpu.make_async_copy(
            edge_attr_hbm.at[pl.ds(0, CA * MAX_NB)], rows_v.at[slot],
            sems[slot]).wait()

        @pl.when(i >= 2)
        def _():
            pltpu.make_async_copy(
                acc_v.at[slot], amsg_hbm.at[pl.ds(0, CA)], osems[slot]).wait()

        def atom(a, c2):
            r0 = a * MAX_NB
            for col in range(D // 16):
                s = pl.ds(col * 16, 16)
                acc = rows_v[slot, r0, s]
                for nb in range(1, MAX_NB):
                    acc = acc + rows_v[slot, r0 + nb, s]
                acc_v[slot, a, s] = acc
            return c2

        lax.fori_loop(0, CA, atom, 0, unroll=False)
        pltpu.async_copy(acc_v.at[slot],
                         amsg_hbm.at[pl.ds(abase + i * CA, CA)], osems[slot])

    _pipelined(nchunks, issue, process)
    pltpu.make_async_copy(acc_v.at[0], amsg_hbm.at[pl.ds(0, CA)], osem0).wait()
    pltpu.make_async_copy(acc_v.at[1], amsg_hbm.at[pl.ds(0, CA)], osem1).wait()


@functools.partial(
    pl.kernel,
    out_type=jax.ShapeDtypeStruct((E, D), jnp.float32),
    mesh=_MESH,
    scratch_types=[
        pltpu.VMEM((EPW,), jnp.int32),
        pltpu.VMEM((EPW,), jnp.int32),
        pltpu.VMEM((2, CE, D), jnp.float32),
        pltpu.VMEM((2, CE, D), jnp.float32),
        pltpu.VMEM((2, CE, D), jnp.float32),
        pltpu.SemaphoreType.DMA,
        pltpu.SemaphoreType.DMA,
        pltpu.SemaphoreType.DMA,
        pltpu.SemaphoreType.DMA,
    ],
)
def _message(b2a_hbm, b2revb_hbm, amsg_hbm, edge_attr_hbm, out_hbm,
             idxa_v, idxr_v, ra_v, rr_v, ro_v, sem0, sem1, osem0, osem1):
    wid = _wid()
    ebase = wid * EPW
    pltpu.sync_copy(b2a_hbm.at[pl.ds(ebase, EPW)], idxa_v)
    pltpu.sync_copy(b2revb_hbm.at[pl.ds(ebase, EPW)], idxr_v)
    sems = (sem0, sem1)
    osems = (osem0, osem1)
    nchunks = EPW // CE

    def issue(i, slot):
        e0 = i * CE
        pltpu.async_copy(amsg_hbm.at[idxa_v.at[pl.ds(e0, CE)]],
                         ra_v.at[slot], sems[slot])
        pltpu.async_copy(edge_attr_hbm.at[idxr_v.at[pl.ds(e0, CE)]],
                         rr_v.at[slot], sems[slot])

    def process(i, slot):
        pltpu.make_async_copy(
            amsg_hbm.at[pl.ds(0, CE)], ra_v.at[slot], sems[slot]).wait()
        pltpu.make_async_copy(
            edge_attr_hbm.at[pl.ds(0, CE)], rr_v.at[slot], sems[slot]).wait()

        @pl.when(i >= 2)
        def _():
            pltpu.make_async_copy(
                ro_v.at[slot], out_hbm.at[pl.ds(0, CE)], osems[slot]).wait()

        def row(r, c2):
            for col in range(D // 16):
                s = pl.ds(col * 16, 16)
                ro_v[slot, r, s] = ra_v[slot, r, s] - rr_v[slot, r, s]
            return c2

        lax.fori_loop(0, CE, row, 0, unroll=False)
        pltpu.async_copy(ro_v.at[slot],
                         out_hbm.at[pl.ds(ebase + i * CE, CE)], osems[slot])

    _pipelined(nchunks, issue, process)
    pltpu.make_async_copy(ro_v.at[0], out_hbm.at[pl.ds(0, CE)], osem0).wait()
    pltpu.make_async_copy(ro_v.at[1], out_hbm.at[pl.ds(0, CE)], osem1).wait()


BE = 2000


def _mm_body(x_ref, w_ref, b_ref, o_ref):
    x = x_ref[...].astype(jnp.bfloat16)
    o_ref[...] = lax.dot_general(
        x, w_ref[...], (((1,), (0,)), ((), ())),
        preferred_element_type=jnp.float32) + b_ref[...]


def _linear(message, wt_bf16, bias):
    return pl.pallas_call(
        _mm_body,
        grid=(E // BE,),
        in_specs=[
            pl.BlockSpec((BE, D), lambda i: (i, 0)),
            pl.BlockSpec((D, D), lambda i: (0, 0)),
            pl.BlockSpec((1, D), lambda i: (0, 0)),
        ],
        out_specs=pl.BlockSpec((BE, D), lambda i: (i, 0)),
        out_shape=jax.ShapeDtypeStruct((E, D), jnp.float32),
    )(message, wt_bf16, bias)


def kernel(x, edge_index, edge_attr, a2b, b2a, b2revb, W, b):
    del x, edge_index
    a2b_flat = jnp.pad(a2b.reshape(-1), (0, (N_PAD - N) * MAX_NB))
    amsg = _aggregate(a2b_flat, edge_attr)
    message = _message(b2a, b2revb, amsg, edge_attr)
    wt = W.T.astype(jnp.bfloat16)
    return _linear(message, wt, b.reshape(1, D))
